# baseline (speedup 1.0000x reference)
import functools

import jax
import jax.numpy as jnp
import numpy as np
from jax import lax
from jax.experimental import pallas as pl
from jax.experimental.pallas import tpu as pltpu
from jax.experimental.pallas import tpu_sc as plsc

F32 = jnp.float32
I32 = jnp.int32

N = 10000
D = 256
NC, NS, L = 2, 16, 16
NW = NC * NS
RPW = 320
NPAD = NW * RPW
ACC = RPW * D
CAP = 12288
CAPP = CAP + 128
EG = 16
BIG = 3.0e38
CH = 4000
DP = D // 2
ACC2 = RPW * DP
BF = jnp.bfloat16
_BIGBF_BITS = int(np.float32(BIG).view(np.uint32) >> 16)
BIG2 = np.int32(np.uint32((_BIGBF_BITS << 16) | _BIGBF_BITS))

_MESH = plsc.VectorSubcoreMesh(core_axis_name="c", subcore_axis_name="s")
_SC_PARAMS = pltpu.CompilerParams(needs_layout_passes=False)


def _wid():
    return lax.axis_index("s") * NC + lax.axis_index("c")


NBIN = 336
_RC_BASE = 1


def _filter_one(E, dst_hbm, src_hbm, srcl_out, dstl_out, cnt_out,
                dstc, srcc, srclv, dstlv, srcs, dsts, base2, cntv, w, lo, hi):
        n_chunks = E // CH

        def chunk(g, count):
            pltpu.sync_copy(dst_hbm.at[pl.ds(g * CH, CH)], dstc)
            pltpu.sync_copy(src_hbm.at[pl.ds(g * CH, CH)], srcc)

            def grp(i, count):
                d16 = dstc[pl.ds(i * L, L)]
                s16 = srcc[pl.ds(i * L, L)]
                m = (d16 >= lo) & (d16 < hi)
                c = plsc.cumsum(m.astype(I32))
                pos = count - 1 + c
                plsc.store_scatter(srclv, [pos], s16, mask=m)
                plsc.store_scatter(dstlv, [pos], d16 - lo, mask=m)
                return count + plsc.all_reduce_population_count(m)[0]

            return lax.fori_loop(0, CH // L, grp, count)

        count = lax.fori_loop(0, n_chunks, chunk, jnp.int32(0))

        sent_d = jnp.full((L,), RPW, I32)
        sent_s = jnp.zeros((L,), I32)
        for k in range(8):
            dstlv[pl.ds(count + k * L, L)] = sent_d
            srclv[pl.ds(count + k * L, L)] = sent_s
        npad = jnp.maximum(jnp.int32(128), ((count + 127) // 128) * 128)
        ngrp = npad // L

        zeros = jnp.zeros((L,), I32)
        for c in range(NBIN // L):
            base2[pl.ds(c * L, L)] = zeros

        def hist(i, _):
            d16 = dstlv[pl.ds(i * L, L)]
            rc, ml = plsc.scan_count(d16)
            tot = rc + (1 - _RC_BASE)
            plsc.addupdate_scatter(base2, [d16], tot, mask=ml)
            return 0

        lax.fori_loop(0, ngrp, hist, 0)

        carry = jnp.int32(0)
        for c in range(NBIN // L):
            v = base2[pl.ds(c * L, L)]
            cs = plsc.cumsum(v)
            base2[pl.ds(c * L, L)] = cs - v + carry
            carry = carry + jnp.max(cs)

        def perm(i, _):
            d16 = dstlv[pl.ds(i * L, L)]
            s16 = srclv[pl.ds(i * L, L)]
            rc, ml = plsc.scan_count(d16)
            r0 = rc - _RC_BASE
            b16 = plsc.load_gather(base2, [d16])
            pos = b16 + r0
            plsc.store_scatter(base2, [d16], pos + 1, mask=ml)
            plsc.store_scatter(srcs, [pos], s16)
            plsc.store_scatter(dsts, [pos], d16)
            return 0

        lax.fori_loop(0, ngrp, perm, 0)

        cntv[...] = jnp.full((L,), npad, I32)
        pltpu.sync_copy(srcs, srcl_out.at[w])
        pltpu.sync_copy(dsts, dstl_out.at[w])
        pltpu.sync_copy(cntv, cnt_out.at[w])


def _make_filter(E):
    def body(dst_hbm, src_hbm, srcl_out, dstl_out, cnt_out,
             dstc, srcc, srclv, dstlv, srcs, dsts, base2, cntv):
        w = _wid()
        _filter_one(E, dst_hbm, src_hbm, srcl_out, dstl_out, cnt_out,
                    dstc, srcc, srclv, dstlv, srcs, dsts, base2, cntv,
                    w, w * RPW, w * RPW + RPW)

    return pl.kernel(
        body,
        out_type=(jax.ShapeDtypeStruct((NW, CAPP), I32),
                  jax.ShapeDtypeStruct((NW, CAPP), I32),
                  jax.ShapeDtypeStruct((NW, L), I32)),
        mesh=_MESH,
        compiler_params=_SC_PARAMS,
        scratch_types=[pltpu.VMEM((CH,), I32),
                       pltpu.VMEM((CH,), I32),
                       pltpu.VMEM((CAPP,), I32),
                       pltpu.VMEM((CAPP,), I32),
                       pltpu.VMEM((CAPP,), I32),
                       pltpu.VMEM((CAPP,), I32),
                       pltpu.VMEM((NBIN,), I32),
                       pltpu.VMEM((L,), I32)],
    )


NBUF = 8


def _segmin_body(table, srcl, dstl, cnt, seg_out,
                 acc, srclv, dstlv, cntv, rows, sem0, sem1, sem2, sem3,
                 sem4, sem5, sem6, sem7):
    w = _wid()
    pltpu.sync_copy(cnt.at[w], cntv)
    pltpu.sync_copy(srcl.at[w], srclv)
    pltpu.sync_copy(dstl.at[w], dstlv)
    n = jnp.max(cntv[...])

    big2 = jnp.full((L,), BIG2, I32)

    def ini(i, _):
        for k in range(DP // L):
            acc[pl.ds(i * DP + k * L, L)] = big2
        return 0

    lax.fori_loop(0, RPW + 1, ini, 0)

    sems = (sem0, sem1, sem2, sem3, sem4, sem5, sem6, sem7)
    for b in range(NBUF):
        pltpu.async_copy(table.at[srclv.at[pl.ds(b * EG, EG)]],
                         rows.at[b], sems[b])

    ngrp = n // EG
    nquad = n // (NBUF * EG)
    bigbf = plsc.bitcast(jnp.full((L,), BIG2, I32), BF)
    nch = DP // L

    def quad(q, carry):
        for b in range(NBUF):
            g = NBUF * q + b
            pltpu.make_async_copy(table.at[srclv.at[pl.ds(0, EG)]],
                                  rows.at[b], sems[b]).wait()
            dvec = dstlv[pl.ds(g * EG, L)] * DP
            for j in range(L):
                off = dvec[j]
                cur = carry[0]

                def flush(cur=cur, regs=carry[1:]):
                    for k in range(nch):
                        acc[pl.ds(cur + k * L, L)] = plsc.bitcast(
                            regs[k], I32)
                    return (bigbf,) * nch

                def keep(regs=carry[1:]):
                    return tuple(regs)

                regs = lax.cond(off != cur, flush, keep)
                rcs = [plsc.bitcast(rows[b, j, pl.ds(k * L, L)], BF)
                       for k in range(nch)]
                carry = (off,) + tuple(
                    jnp.minimum(a, r) for a, r in zip(regs, rcs))

            @pl.when(g + NBUF < ngrp)
            def _(g=g, b=b):
                pltpu.async_copy(
                    table.at[srclv.at[pl.ds((g + NBUF) * EG, EG)]],
                    rows.at[b], sems[b])
        return carry

    carry0 = (jnp.int32(ACC2),) + (bigbf,) * nch
    carry = lax.fori_loop(0, nquad, quad, carry0)
    for k in range(nch):
        acc[pl.ds(carry[0] + k * L, L)] = plsc.bitcast(carry[1 + k], I32)
    pltpu.sync_copy(acc.at[pl.ds(0, ACC2)], seg_out.at[w])


_SEGMIN = pl.kernel(
    _segmin_body,
    out_type=jax.ShapeDtypeStruct((NW, ACC2), I32),
    mesh=_MESH,
    compiler_params=_SC_PARAMS,
    scratch_types=[pltpu.VMEM((ACC2 + DP,), I32),
                   pltpu.VMEM((CAPP,), I32),
                   pltpu.VMEM((CAPP,), I32),
                   pltpu.VMEM((L,), I32),
                   pltpu.VMEM((NBUF, EG, DP), I32),
                   pltpu.SemaphoreType.DMA,
                   pltpu.SemaphoreType.DMA,
                   pltpu.SemaphoreType.DMA,
                   pltpu.SemaphoreType.DMA,
                   pltpu.SemaphoreType.DMA,
                   pltpu.SemaphoreType.DMA,
                   pltpu.SemaphoreType.DMA,
                   pltpu.SemaphoreType.DMA],
)


BLKE = 1024
BLKM = 512
U32 = jnp.uint32


def _pack_tc(y):
    u = lax.bitcast_convert_type(y, U32)
    r = u + jnp.uint32(0x7FFF) + ((u >> 16) & jnp.uint32(1))
    h = r >> 16
    lo = h[:, :DP]
    hi = h[:, DP:]
    return lax.bitcast_convert_type(lo | (hi << 16), I32)


def _unpack_tc(w):
    wu = lax.bitcast_convert_type(w, U32)
    lo = lax.bitcast_convert_type(wu << 16, F32)
    hi = lax.bitcast_convert_type(wu & jnp.uint32(0xFFFF0000), F32)
    return lo, hi


def _enc_body(v_ref, e_ref, f_ref, wv_ref, we_ref, wf_ref,
              bv_ref, be_ref, bf_ref, xv_ref, xe_ref, xf_ref,
              bv16_ref, be16_ref, bf16_ref):
    for x_ref, w_ref, b_ref, o_ref, o2_ref in (
            (v_ref, wv_ref, bv_ref, xv_ref, bv16_ref),
            (e_ref, we_ref, be_ref, xe_ref, be16_ref),
            (f_ref, wf_ref, bf_ref, xf_ref, bf16_ref)):
        y = jnp.dot(x_ref[...], w_ref[...], preferred_element_type=F32)
        y = y + b_ref[...]
        y = jnp.where(y >= 0, y, 0.01 * y)
        y = jnp.where(jnp.isnan(y), 0.0, y)
        y = jnp.clip(y, -10000.0, 10000.0)
        o_ref[...] = y
        o2_ref[...] = _pack_tc(y)


def _encoders(v8, e8, f8, wv, we, wf, bv, be, bf):
    row = pl.BlockSpec((BLKE, 8), lambda i: (i, 0))
    full = pl.BlockSpec((8, D), lambda i: (0, 0))
    bias = pl.BlockSpec((1, D), lambda i: (0, 0))
    out = pl.BlockSpec((BLKE, D), lambda i: (i, 0))
    outp = pl.BlockSpec((BLKE, DP), lambda i: (i, 0))
    return pl.pallas_call(
        _enc_body,
        grid=(NPAD // BLKE,),
        in_specs=[row, row, row, full, full, full, bias, bias, bias],
        out_specs=[out, out, out, outp, outp, outp],
        out_shape=[jax.ShapeDtypeStruct((NPAD, D), F32)] * 3
        + [jax.ShapeDtypeStruct((NPAD, DP), I32)] * 3,
    )(v8, e8, f8, wv, we, wf, bv, be, bf)


def _make_mlp(self_loop):
    def body(x_ref, s_ref, w0_ref, w1a_ref, w1b_ref, b_ref, o_ref, o2_ref):
        x = x_ref[...]
        s_lo, s_hi = _unpack_tc(s_ref[...])

        def mk_mx(xh, sh):
            if self_loop:
                return xh - jnp.minimum(sh, xh)
            return jnp.where(sh > 1e30, 0.0, xh - sh)

        mx_lo = mk_mx(x[:, :DP], s_lo)
        mx_hi = mk_mx(x[:, DP:], s_hi)
        y = jnp.dot(x, w0_ref[...], preferred_element_type=F32)
        y = y + jnp.dot(mx_lo, w1a_ref[...], preferred_element_type=F32)
        y = y + jnp.dot(mx_hi, w1b_ref[...], preferred_element_type=F32)
        y = y + b_ref[...]
        y = jnp.where(y >= 0, y, 0.01 * y)
        y = x + y
        y = jnp.where(jnp.isnan(y), 0.0, y)
        y = jnp.clip(y, -10000.0, 10000.0)
        o_ref[...] = y
        o2_ref[...] = _pack_tc(y)

    row = pl.BlockSpec((BLKM, D), lambda i: (i, 0))
    rowp = pl.BlockSpec((BLKM, DP), lambda i: (i, 0))
    wsp = pl.BlockSpec((D, D), lambda i: (0, 0))
    wsph = pl.BlockSpec((DP, D), lambda i: (0, 0))
    bias = pl.BlockSpec((1, D), lambda i: (0, 0))

    def run(x, s, w, b):
        return pl.pallas_call(
            body,
            grid=(NPAD // BLKM,),
            in_specs=[row, rowp, wsp, wsph, wsph, bias],
            out_specs=[row, rowp],
            out_shape=[jax.ShapeDtypeStruct((NPAD, D), F32),
                       jax.ShapeDtypeStruct((NPAD, DP), I32)],
        )(x, s, w[:D], w[D:D + DP], w[D + DP:], b.reshape(1, D))

    return run


_MLP_PLAIN = _make_mlp(False)
_MLP_LOOP = _make_mlp(True)


def _pad_feat(x):
    out = jnp.zeros((NPAD, 8), F32)
    return out.at[:x.shape[0], :x.shape[1]].set(x)


def _pad_w(w):
    out = jnp.zeros((8, D), F32)
    return out.at[:w.shape[0]].set(w)


def kernel(vertices, edges, faces, edge_to_vertex, face_to_edge, face_to_face,
           Wv, bv, We, be, Wf, bf, Wv2e, bv2e, We2f, be2f,
           Wm0, bm0, Wm1, bm1, Wm2, bm2):
    x_v, x_e, x_f, xbf_v, xbf_e, xbf_f = _encoders(
        _pad_feat(vertices), _pad_feat(edges), _pad_feat(faces),
        _pad_w(Wv), _pad_w(We), _pad_w(Wf),
        bv.reshape(1, D), be.reshape(1, D), bf.reshape(1, D))

    ev = _make_filter(edge_to_vertex.shape[1])(
        edge_to_vertex[0], edge_to_vertex[1])
    fe = _make_filter(face_to_edge.shape[1])(
        face_to_edge[0], face_to_edge[1])
    ff = _make_filter(face_to_face.shape[1])(
        face_to_face[1], face_to_face[0])

    def seg(xp, lists):
        return _SEGMIN(xp, *lists).reshape(NPAD, DP)

    x_e, xbf_e = _MLP_PLAIN(x_e, seg(xbf_v, ev), Wv2e, bv2e)
    x_f, xbf_f = _MLP_PLAIN(x_f, seg(xbf_e, fe), We2f, be2f)
    for w, b in ((Wm0, bm0), (Wm1, bm1), (Wm2, bm2)):
        x_f, xbf_f = _MLP_LOOP(x_f, seg(xbf_f, ff), w, b)
    return x_f[:N]


# speedup vs baseline: 1.2794x; 1.2794x over previous
import jax
import jax.numpy as jnp
import numpy as np
from jax import lax
from jax.experimental import pallas as pl
from jax.experimental.pallas import tpu as pltpu
from jax.experimental.pallas import tpu_sc as plsc

F32 = jnp.float32
I32 = jnp.int32

N = 10000
D = 256
NC, NS, L = 2, 16, 16
NW = NC * NS
RPW = 320
NPAD = NW * RPW
ACC = RPW * D
CAP = 12288
CAPP = CAP + 64
EG = 16
BIG = 3.0e38
CH = 4000
DP = D // 2
ACC2 = RPW * DP
BF = jnp.bfloat16
_BIGBF_BITS = int(np.float32(BIG).view(np.uint32) >> 16)
BIG2 = np.int32(np.uint32((_BIGBF_BITS << 16) | _BIGBF_BITS))

_MESH = plsc.VectorSubcoreMesh(core_axis_name="c", subcore_axis_name="s")
_SC_PARAMS = pltpu.CompilerParams(needs_layout_passes=False)


def _wid():
    return lax.axis_index("s") * NC + lax.axis_index("c")


NBIN = 336
_RC_BASE = 1


def _filter_one(E, dst_hbm, src_hbm, srcl_out, dstl_out, cnt_out,
                dstc, srcc, srclv, dstlv, srcs, dsts, base2, cntv, w, lo, hi):
        n_chunks = E // CH

        def chunk(g, count):
            pltpu.sync_copy(dst_hbm.at[pl.ds(g * CH, CH)], dstc)
            pltpu.sync_copy(src_hbm.at[pl.ds(g * CH, CH)], srcc)

            def grp(i, count):
                d16 = dstc[pl.ds(i * L, L)]
                s16 = srcc[pl.ds(i * L, L)]
                m = (d16 >= lo) & (d16 < hi)
                c = plsc.cumsum(m.astype(I32))
                pos = count - 1 + c
                plsc.store_scatter(srclv, [pos], s16, mask=m)
                plsc.store_scatter(dstlv, [pos], d16 - lo, mask=m)
                return count + plsc.all_reduce_population_count(m)[0]

            return lax.fori_loop(0, CH // L, grp, count)

        count = lax.fori_loop(0, n_chunks, chunk, jnp.int32(0))

        sent_d = jnp.full((L,), RPW, I32)
        sent_s = jnp.zeros((L,), I32)
        for k in range(4):
            dstlv[pl.ds(count + k * L, L)] = sent_d
            srclv[pl.ds(count + k * L, L)] = sent_s
        npad = jnp.maximum(jnp.int32(64), ((count + 63) // 64) * 64)
        ngrp = npad // L

        zeros = jnp.zeros((L,), I32)
        for c in range(NBIN // L):
            base2[pl.ds(c * L, L)] = zeros

        def hist(i, _):
            d16 = dstlv[pl.ds(i * L, L)]
            rc, ml = plsc.scan_count(d16)
            tot = rc + (1 - _RC_BASE)
            plsc.addupdate_scatter(base2, [d16], tot, mask=ml)
            return 0

        lax.fori_loop(0, ngrp, hist, 0)

        carry = jnp.int32(0)
        for c in range(NBIN // L):
            v = base2[pl.ds(c * L, L)]
            cs = plsc.cumsum(v)
            base2[pl.ds(c * L, L)] = cs - v + carry
            carry = carry + jnp.max(cs)

        def perm(i, _):
            d16 = dstlv[pl.ds(i * L, L)]
            s16 = srclv[pl.ds(i * L, L)]
            rc, ml = plsc.scan_count(d16)
            r0 = rc - _RC_BASE
            b16 = plsc.load_gather(base2, [d16])
            pos = b16 + r0
            plsc.store_scatter(base2, [d16], pos + 1, mask=ml)
            plsc.store_scatter(srcs, [pos], s16)
            plsc.store_scatter(dsts, [pos], d16)
            return 0

        lax.fori_loop(0, ngrp, perm, 0)

        cntv[...] = jnp.full((L,), npad, I32)
        pltpu.sync_copy(srcs, srcl_out.at[w])
        pltpu.sync_copy(dsts, dstl_out.at[w])
        pltpu.sync_copy(cntv, cnt_out.at[w])


def _make_filter(E):
    def body(dst_hbm, src_hbm, srcl_out, dstl_out, cnt_out,
             dstc, srcc, srclv, dstlv, srcs, dsts, base2, cntv):
        w = _wid()
        _filter_one(E, dst_hbm, src_hbm, srcl_out, dstl_out, cnt_out,
                    dstc, srcc, srclv, dstlv, srcs, dsts, base2, cntv,
                    w, w * RPW, w * RPW + RPW)

    return pl.kernel(
        body,
        out_type=(jax.ShapeDtypeStruct((NW, CAPP), I32),
                  jax.ShapeDtypeStruct((NW, CAPP), I32),
                  jax.ShapeDtypeStruct((NW, L), I32)),
        mesh=_MESH,
        compiler_params=_SC_PARAMS,
        scratch_types=[pltpu.VMEM((CH,), I32),
                       pltpu.VMEM((CH,), I32),
                       pltpu.VMEM((CAPP,), I32),
                       pltpu.VMEM((CAPP,), I32),
                       pltpu.VMEM((CAPP,), I32),
                       pltpu.VMEM((CAPP,), I32),
                       pltpu.VMEM((NBIN,), I32),
                       pltpu.VMEM((L,), I32)],
    )


NBUF = 4


def _segmin_body(table, srcl, dstl, cnt, seg_out,
                 acc, srclv, dstlv, cntv, rows, sem0, sem1, sem2, sem3):
    w = _wid()
    pltpu.sync_copy(cnt.at[w], cntv)
    pltpu.sync_copy(srcl.at[w], srclv)
    pltpu.sync_copy(dstl.at[w], dstlv)
    n = jnp.max(cntv[...])

    big2 = jnp.full((L,), BIG2, I32)

    def ini(i, _):
        for k in range(DP // L):
            acc[pl.ds(i * DP + k * L, L)] = big2
        return 0

    lax.fori_loop(0, RPW + 1, ini, 0)

    sems = (sem0, sem1, sem2, sem3)
    for b in range(NBUF):
        pltpu.async_copy(table.at[srclv.at[pl.ds(b * EG, EG)]],
                         rows.at[b], sems[b])

    ngrp = n // EG
    nquad = n // (NBUF * EG)
    bigbf = plsc.bitcast(jnp.full((L,), BIG2, I32), BF)
    nch = DP // L

    def quad(q, carry):
        for b in range(NBUF):
            g = NBUF * q + b
            pltpu.make_async_copy(table.at[srclv.at[pl.ds(0, EG)]],
                                  rows.at[b], sems[b]).wait()
            dvec = dstlv[pl.ds(g * EG, L)] * DP
            for j in range(L):
                off = dvec[j]
                cur = carry[0]

                def flush(cur=cur, regs=carry[1:]):
                    for k in range(nch):
                        acc[pl.ds(cur + k * L, L)] = plsc.bitcast(
                            regs[k], I32)
                    return (bigbf,) * nch

                def keep(regs=carry[1:]):
                    return tuple(regs)

                regs = lax.cond(off != cur, flush, keep)
                rcs = [plsc.bitcast(rows[b, j, pl.ds(k * L, L)], BF)
                       for k in range(nch)]
                carry = (off,) + tuple(
                    jnp.minimum(a, r) for a, r in zip(regs, rcs))

            @pl.when(g + NBUF < ngrp)
            def _(g=g, b=b):
                pltpu.async_copy(
                    table.at[srclv.at[pl.ds((g + NBUF) * EG, EG)]],
                    rows.at[b], sems[b])
        return carry

    carry0 = (jnp.int32(ACC2),) + (bigbf,) * nch
    carry = lax.fori_loop(0, nquad, quad, carry0)
    for k in range(nch):
        acc[pl.ds(carry[0] + k * L, L)] = plsc.bitcast(carry[1 + k], I32)
    pltpu.sync_copy(acc.at[pl.ds(0, ACC2)], seg_out.at[w])


_SEGMIN = pl.kernel(
    _segmin_body,
    out_type=jax.ShapeDtypeStruct((NW, ACC2), I32),
    mesh=_MESH,
    compiler_params=_SC_PARAMS,
    scratch_types=[pltpu.VMEM((ACC2 + DP,), I32),
                   pltpu.VMEM((CAPP,), I32),
                   pltpu.VMEM((CAPP,), I32),
                   pltpu.VMEM((L,), I32),
                   pltpu.VMEM((NBUF, EG, DP), I32),
                   pltpu.SemaphoreType.DMA,
                   pltpu.SemaphoreType.DMA,
                   pltpu.SemaphoreType.DMA,
                   pltpu.SemaphoreType.DMA],
)


BLKE = 1024
BLKM = 512
U32 = jnp.uint32


def _pack_tc(y):
    u = lax.bitcast_convert_type(y, U32)
    r = u + jnp.uint32(0x7FFF) + ((u >> 16) & jnp.uint32(1))
    h = r >> 16
    lo = h[:, :DP]
    hi = h[:, DP:]
    return lax.bitcast_convert_type(lo | (hi << 16), I32)


def _unpack_tc(w):
    wu = lax.bitcast_convert_type(w, U32)
    lo = lax.bitcast_convert_type(wu << 16, F32)
    hi = lax.bitcast_convert_type(wu & jnp.uint32(0xFFFF0000), F32)
    return lo, hi


def _enc_body(v_ref, e_ref, f_ref, wv_ref, we_ref, wf_ref,
              bv_ref, be_ref, bf_ref, xv_ref, xe_ref, xf_ref,
              bv16_ref, be16_ref, bf16_ref):
    for x_ref, w_ref, b_ref, o_ref, o2_ref in (
            (v_ref, wv_ref, bv_ref, xv_ref, bv16_ref),
            (e_ref, we_ref, be_ref, xe_ref, be16_ref),
            (f_ref, wf_ref, bf_ref, xf_ref, bf16_ref)):
        y = jnp.dot(x_ref[...], w_ref[...], preferred_element_type=F32)
        y = y + b_ref[...]
        y = jnp.where(y >= 0, y, 0.01 * y)
        y = jnp.where(jnp.isnan(y), 0.0, y)
        y = jnp.clip(y, -10000.0, 10000.0)
        o_ref[...] = y
        o2_ref[...] = _pack_tc(y)


def _encoders(v8, e8, f8, wv, we, wf, bv, be, bf):
    row = pl.BlockSpec((BLKE, 8), lambda i: (i, 0))
    full = pl.BlockSpec((8, D), lambda i: (0, 0))
    bias = pl.BlockSpec((1, D), lambda i: (0, 0))
    out = pl.BlockSpec((BLKE, D), lambda i: (i, 0))
    outp = pl.BlockSpec((BLKE, DP), lambda i: (i, 0))
    return pl.pallas_call(
        _enc_body,
        grid=(NPAD // BLKE,),
        in_specs=[row, row, row, full, full, full, bias, bias, bias],
        out_specs=[out, out, out, outp, outp, outp],
        out_shape=[jax.ShapeDtypeStruct((NPAD, D), F32)] * 3
        + [jax.ShapeDtypeStruct((NPAD, DP), I32)] * 3,
    )(v8, e8, f8, wv, we, wf, bv, be, bf)


def _make_mlp(self_loop):
    def body(x_ref, s_ref, w0_ref, w1a_ref, w1b_ref, b_ref, o_ref, o2_ref):
        x = x_ref[...]
        s_lo, s_hi = _unpack_tc(s_ref[...])

        def mk_mx(xh, sh):
            if self_loop:
                return xh - jnp.minimum(sh, xh)
            return jnp.where(sh > 1e30, 0.0, xh - sh)

        mx_lo = mk_mx(x[:, :DP], s_lo)
        mx_hi = mk_mx(x[:, DP:], s_hi)
        y = jnp.dot(x, w0_ref[...], preferred_element_type=F32)
        y = y + jnp.dot(mx_lo, w1a_ref[...], preferred_element_type=F32)
        y = y + jnp.dot(mx_hi, w1b_ref[...], preferred_element_type=F32)
        y = y + b_ref[...]
        y = jnp.where(y >= 0, y, 0.01 * y)
        y = x + y
        y = jnp.where(jnp.isnan(y), 0.0, y)
        y = jnp.clip(y, -10000.0, 10000.0)
        o_ref[...] = y
        o2_ref[...] = _pack_tc(y)

    row = pl.BlockSpec((BLKM, D), lambda i: (i, 0))
    rowp = pl.BlockSpec((BLKM, DP), lambda i: (i, 0))
    wsp = pl.BlockSpec((D, D), lambda i: (0, 0))
    wsph = pl.BlockSpec((DP, D), lambda i: (0, 0))
    bias = pl.BlockSpec((1, D), lambda i: (0, 0))

    def run(x, s, w, b):
        return pl.pallas_call(
            body,
            grid=(NPAD // BLKM,),
            in_specs=[row, rowp, wsp, wsph, wsph, bias],
            out_specs=[row, rowp],
            out_shape=[jax.ShapeDtypeStruct((NPAD, D), F32),
                       jax.ShapeDtypeStruct((NPAD, DP), I32)],
        )(x, s, w[:D], w[D:D + DP], w[D + DP:], b.reshape(1, D))

    return run


_MLP_PLAIN = _make_mlp(False)
_MLP_LOOP = _make_mlp(True)


def _pad_feat(x):
    out = jnp.zeros((NPAD, 8), F32)
    return out.at[:x.shape[0], :x.shape[1]].set(x)


def _pad_w(w):
    out = jnp.zeros((8, D), F32)
    return out.at[:w.shape[0]].set(w)


def kernel(vertices, edges, faces, edge_to_vertex, face_to_edge, face_to_face,
           Wv, bv, We, be, Wf, bf, Wv2e, bv2e, We2f, be2f,
           Wm0, bm0, Wm1, bm1, Wm2, bm2):
    x_v, x_e, x_f, xbf_v, xbf_e, xbf_f = _encoders(
        _pad_feat(vertices), _pad_feat(edges), _pad_feat(faces),
        _pad_w(Wv), _pad_w(We), _pad_w(Wf),
        bv.reshape(1, D), be.reshape(1, D), bf.reshape(1, D))

    ev = _make_filter(edge_to_vertex.shape[1])(
        edge_to_vertex[0], edge_to_vertex[1])
    fe = _make_filter(face_to_edge.shape[1])(
        face_to_edge[0], face_to_edge[1])
    ff = _make_filter(face_to_face.shape[1])(
        face_to_face[1], face_to_face[0])

    def seg(xp, lists):
        return _SEGMIN(xp, *lists).reshape(NPAD, DP)

    x_e, xbf_e = _MLP_PLAIN(x_e, seg(xbf_v, ev), Wv2e, bv2e)
    x_f, xbf_f = _MLP_PLAIN(x_f, seg(xbf_e, fe), We2f, be2f)
    for w, b in ((Wm0, bm0), (Wm1, bm1), (Wm2, bm2)):
        x_f, xbf_f = _MLP_LOOP(x_f, seg(xbf_f, ff), w, b)
    return x_f[:N]
